# Initial kernel scaffold; baseline (speedup 1.0000x reference)
import jax
import jax.numpy as jnp
from jax.experimental import pallas as pl


def kernel(x, edge_index, edge_weight, batch, W1, b1, W2, b2, W3, b3, W4, b4, Wl, bl, Wl2, bl2):
    raise NotImplementedError("write your pallas kernel here")


# speedup vs baseline: 12.0492x; 12.0492x over previous
import functools

import jax
import jax.numpy as jnp
from jax import lax
from jax.experimental import pallas as pl
from jax.experimental.pallas import tpu as pltpu
from jax.experimental.pallas import tpu_sc as plsc

N = 50000
E = 800000
H = 64
G = 128
EPS = 1e-5

NC = 2
NS = 16
NW = NC * NS
CW = 128
NCHUNK = (E + CW - 1) // CW
NCHUNK_PAD = ((NCHUNK + NW * 8 - 1) // (NW * 8)) * (NW * 8)
EP = NCHUNK_PAD * CW
DEG_CPT = NCHUNK_PAD // NW
SP_CPT = NCHUNK_PAD // NS
NR = 50048
RPT = 3128
RPT_LAST = N - (NS - 1) * RPT
HF = H // 2
GC = 40

_MESH = plsc.VectorSubcoreMesh(core_axis_name="c", subcore_axis_name="s",
                               num_cores=NC, num_subcores=NS)


def _deg_body(dst_hbm, hist_out, didx, hist):
    cid = lax.axis_index("c")
    sid = lax.axis_index("s")
    wid = sid * NC + cid
    pltpu.sync_copy(dst_hbm.at[pl.ds(wid * DEG_CPT, DEG_CPT)], didx)
    zero16 = jnp.zeros((16,), jnp.float32)

    def zbody(i, c):
        hist[pl.ds(i * 16, 16)] = zero16
        return c

    lax.fori_loop(0, NR // 16, zbody, 0)
    one16 = jnp.ones((16,), jnp.float32)

    def ebody(j, c):
        def ibody(k, c2):
            idx16 = didx[j, pl.ds(k * 16, 16)]
            plsc.addupdate_scatter(hist, [idx16], one16)
            return c2

        return lax.fori_loop(0, CW // 16, ibody, c)

    lax.fori_loop(0, DEG_CPT, ebody, 0)
    pltpu.sync_copy(hist, hist_out.at[wid])


_SC_PARAMS = pltpu.CompilerParams(needs_layout_passes=False,
                                  use_tc_tiling_on_sc=False)

_deg_call = pl.kernel(
    _deg_body,
    out_type=jax.ShapeDtypeStruct((NW, NR), jnp.float32),
    mesh=_MESH,
    compiler_params=_SC_PARAMS,
    scratch_types=[
        pltpu.VMEM((DEG_CPT, CW), jnp.int32),
        pltpu.VMEM((NR,), jnp.float32),
    ],
)


def _spmm_body(u_hbm, src_hbm, dst_hbm, s_out, sidx, didx, rows, acc, gsem):
    cid = lax.axis_index("c")
    sid = lax.axis_index("s")
    rb = sid * RPT
    table = u_hbm.at[cid]

    @pl.when(sid < NS - 1)
    def _():
        pltpu.sync_copy(table.at[pl.ds(rb, RPT)], acc.at[pl.ds(rb, RPT)])

    @pl.when(sid == NS - 1)
    def _():
        pltpu.sync_copy(table.at[pl.ds(rb, RPT_LAST)],
                        acc.at[pl.ds(rb, RPT_LAST)])

    plsc.subcore_barrier()

    def gbody(g, c):
        gb = sid * SP_CPT + g * GC
        pltpu.sync_copy(src_hbm.at[pl.ds(gb, GC)], sidx)
        pltpu.sync_copy(dst_hbm.at[pl.ds(gb, GC)], didx)

        def body(j, c2):
            pltpu.async_copy(table.at[sidx.at[j]], rows, gsem).wait()
            pltpu.sync_copy(rows, acc.at[didx.at[j]], add=True)
            return c2

        return lax.fori_loop(0, GC, body, c)

    lax.fori_loop(0, SP_CPT // GC, gbody, 0)
    plsc.subcore_barrier()

    @pl.when(sid < NS - 1)
    def _():
        pltpu.sync_copy(acc.at[pl.ds(rb, RPT)], s_out.at[cid, pl.ds(rb, RPT)])

    @pl.when(sid == NS - 1)
    def _():
        pltpu.sync_copy(acc.at[pl.ds(rb, RPT_LAST)],
                        s_out.at[cid, pl.ds(rb, RPT_LAST)])


_spmm_call = pl.kernel(
    _spmm_body,
    out_type=jax.ShapeDtypeStruct((NC, N, HF), jnp.float32),
    mesh=_MESH,
    compiler_params=_SC_PARAMS,
    scratch_types=[
        pltpu.VMEM((GC, CW), jnp.int32),
        pltpu.VMEM((GC, CW), jnp.int32),
        pltpu.VMEM((CW, HF), jnp.float32),
        pltpu.VMEM_SHARED((NR, HF), jnp.float32),
        pltpu.SemaphoreType.DMA,
    ],
)


def _dinv_body(hist_ref, out_ref):
    s = jnp.sum(hist_ref[...], axis=0)
    out_ref[...] = lax.rsqrt(s[:N] + 1.0)[:, None]


_dinv_call = pl.pallas_call(
    _dinv_body,
    out_shape=jax.ShapeDtypeStruct((N, 1), jnp.float32),
)

R = 2000
NB = N // R


def _pre_body(x_ref, w_ref, dinv_ref, u_ref):
    z = jnp.dot(x_ref[...], w_ref[...], preferred_element_type=jnp.float32)
    u = dinv_ref[...] * z
    u_ref[0, :, :] = u[:, :HF]
    u_ref[1, :, :] = u[:, HF:]


_pre_call = pl.pallas_call(
    _pre_body,
    grid=(NB,),
    in_specs=[
        pl.BlockSpec((R, 10), lambda i: (i, 0)),
        pl.BlockSpec((10, H), lambda i: (0, 0)),
        pl.BlockSpec((R, 1), lambda i: (i, 0)),
    ],
    out_specs=pl.BlockSpec((NC, R, HF), lambda i: (0, i, 0)),
    out_shape=jax.ShapeDtypeStruct((NC, N, HF), jnp.float32),
)


def _mid_body(s_ref, dinv_ref, b_ref, w_ref, u_ref):
    dinv = dinv_ref[...]
    h0 = jnp.maximum(dinv * s_ref[0, :, :] + b_ref[0, :HF], 0.0)
    h1 = jnp.maximum(dinv * s_ref[1, :, :] + b_ref[0, HF:], 0.0)
    z = jnp.dot(h0, w_ref[:HF, :], preferred_element_type=jnp.float32)
    z = z + jnp.dot(h1, w_ref[HF:, :], preferred_element_type=jnp.float32)
    u = dinv * z
    u_ref[0, :, :] = u[:, :HF]
    u_ref[1, :, :] = u[:, HF:]


_mid_call = pl.pallas_call(
    _mid_body,
    grid=(NB,),
    in_specs=[
        pl.BlockSpec((NC, R, HF), lambda i: (0, i, 0)),
        pl.BlockSpec((R, 1), lambda i: (i, 0)),
        pl.BlockSpec((1, H), lambda i: (0, 0)),
        pl.BlockSpec((H, H), lambda i: (0, 0)),
    ],
    out_specs=pl.BlockSpec((NC, R, HF), lambda i: (0, i, 0)),
    out_shape=jax.ShapeDtypeStruct((NC, N, HF), jnp.float32),
)


def _final_body(s_ref, dinv_ref, b_ref, batch_ref, wl_ref, bl_ref, wl2_ref,
                bl2_ref, out_ref, pooled, counts):
    i = pl.program_id(0)

    @pl.when(i == 0)
    def _():
        pooled[...] = jnp.zeros_like(pooled)
        counts[...] = jnp.zeros_like(counts)

    dinv = dinv_ref[...]
    h = dinv * jnp.concatenate([s_ref[0, :, :], s_ref[1, :, :]], axis=1)
    h = h + b_ref[...]
    oh = (batch_ref[...] == lax.broadcasted_iota(jnp.int32, (R, G), 1))
    oh = oh.astype(jnp.float32)
    dn = (((0,), (0,)), ((), ()))
    hp = lax.Precision.HIGHEST
    pooled[...] += lax.dot_general(oh, h, dn, precision=hp,
                                   preferred_element_type=jnp.float32)
    counts[...] += lax.dot_general(oh, jnp.ones((R, 1), jnp.float32), dn,
                                   precision=hp,
                                   preferred_element_type=jnp.float32)

    @pl.when(i == NB - 1)
    def _():
        g = pooled[...] / jnp.maximum(counts[...], 1.0)
        y = jnp.dot(g, wl_ref[...], preferred_element_type=jnp.float32)
        y = y + bl_ref[...]
        mu = jnp.mean(y, axis=0, keepdims=True)
        var = jnp.mean((y - mu) ** 2, axis=0, keepdims=True)
        yr = jnp.maximum((y - mu) * lax.rsqrt(var + EPS), 0.0)
        out_ref[...] = (jnp.dot(yr, wl2_ref[...],
                                preferred_element_type=jnp.float32)
                        + bl2_ref[...])


_final_call = pl.pallas_call(
    _final_body,
    grid=(NB,),
    in_specs=[
        pl.BlockSpec((NC, R, HF), lambda i: (0, i, 0)),
        pl.BlockSpec((R, 1), lambda i: (i, 0)),
        pl.BlockSpec((1, H), lambda i: (0, 0)),
        pl.BlockSpec((R, 1), lambda i: (i, 0)),
        pl.BlockSpec((H, 256), lambda i: (0, 0)),
        pl.BlockSpec((1, 256), lambda i: (0, 0)),
        pl.BlockSpec((256, 2), lambda i: (0, 0)),
        pl.BlockSpec((1, 2), lambda i: (0, 0)),
    ],
    out_specs=pl.BlockSpec((G, 2), lambda i: (0, 0)),
    out_shape=jax.ShapeDtypeStruct((G, 2), jnp.float32),
    scratch_shapes=[
        pltpu.VMEM((G, H), jnp.float32),
        pltpu.VMEM((G, 1), jnp.float32),
    ],
)


def kernel(x, edge_index, edge_weight, batch, W1, b1, W2, b2, W3, b3, W4, b4,
           Wl, bl, Wl2, bl2):
    ei = edge_index.astype(jnp.int32)
    src = ei[0]
    dst = ei[1]
    pad = EP - E
    src_p = jnp.concatenate([src, jnp.zeros((pad,), jnp.int32)])
    src_p = src_p.reshape(NCHUNK_PAD, CW)
    dst_p = jnp.concatenate([dst, jnp.full((pad,), N, jnp.int32)])
    dst_p = dst_p.reshape(NCHUNK_PAD, CW)
    batch2 = batch.astype(jnp.int32).reshape(N, 1)

    hist = _deg_call(dst_p)
    dinv = _dinv_call(hist)
    u = _pre_call(x, W1, dinv)
    for b_l, W_next in ((b1, W2), (b2, W3), (b3, W4)):
        s = _spmm_call(u, src_p, dst_p)
        u = _mid_call(s, dinv, b_l.reshape(1, H), W_next)
    s = _spmm_call(u, src_p, dst_p)
    return _final_call(s, dinv, b4.reshape(1, H), batch2, Wl,
                       bl.reshape(1, 256), Wl2, bl2.reshape(1, 2))


# speedup vs baseline: 16.2920x; 1.3521x over previous
import functools

import jax
import jax.numpy as jnp
from jax import lax
from jax.experimental import pallas as pl
from jax.experimental.pallas import tpu as pltpu
from jax.experimental.pallas import tpu_sc as plsc

N = 50000
E = 800000
H = 64
G = 128
EPS = 1e-5

NC = 2
NS = 16
NW = NC * NS
CW = 128
NCHUNK = (E + CW - 1) // CW
NCHUNK_PAD = ((NCHUNK + NW * 8 - 1) // (NW * 8)) * (NW * 8)
EP = NCHUNK_PAD * CW
DEG_CPT = NCHUNK_PAD // NW
SP_CPT = NCHUNK_PAD // NS
NR = 50048
RPT = 3128
RPT_LAST = N - (NS - 1) * RPT
HF = H // 2
GC = 40

_MESH = plsc.VectorSubcoreMesh(core_axis_name="c", subcore_axis_name="s",
                               num_cores=NC, num_subcores=NS)


def _deg_body(dst_hbm, hist_out, didx, hist):
    cid = lax.axis_index("c")
    sid = lax.axis_index("s")
    wid = sid * NC + cid
    pltpu.sync_copy(dst_hbm.at[pl.ds(wid * DEG_CPT, DEG_CPT)], didx)
    zero16 = jnp.zeros((16,), jnp.float32)

    def zbody(i, c):
        hist[pl.ds(i * 16, 16)] = zero16
        return c

    lax.fori_loop(0, NR // 16, zbody, 0)
    one16 = jnp.ones((16,), jnp.float32)

    def ebody(j, c):
        def ibody(k, c2):
            idx16 = didx[j, pl.ds(k * 16, 16)]
            plsc.addupdate_scatter(hist, [idx16], one16)
            return c2

        return lax.fori_loop(0, CW // 16, ibody, c)

    lax.fori_loop(0, DEG_CPT, ebody, 0)
    pltpu.sync_copy(hist, hist_out.at[wid])


_SC_PARAMS = pltpu.CompilerParams(needs_layout_passes=False,
                                  use_tc_tiling_on_sc=False)

_deg_call = pl.kernel(
    _deg_body,
    out_type=jax.ShapeDtypeStruct((NW, NR), jnp.float32),
    mesh=_MESH,
    compiler_params=_SC_PARAMS,
    scratch_types=[
        pltpu.VMEM((DEG_CPT, CW), jnp.int32),
        pltpu.VMEM((NR,), jnp.float32),
    ],
)


def _spmm_body(u_hbm, src_hbm, dst_hbm, s_out, sidx, didx, rows, acc,
               gsem0, gsem1):
    cid = lax.axis_index("c")
    sid = lax.axis_index("s")
    rb = sid * RPT
    table = u_hbm.at[cid]

    @pl.when(sid < NS - 1)
    def _():
        pltpu.sync_copy(table.at[pl.ds(rb, RPT)], acc.at[pl.ds(rb, RPT)])

    @pl.when(sid == NS - 1)
    def _():
        pltpu.sync_copy(table.at[pl.ds(rb, RPT_LAST)],
                        acc.at[pl.ds(rb, RPT_LAST)])

    plsc.subcore_barrier()

    def gbody(g, c):
        gb = sid * SP_CPT + g * GC
        pltpu.sync_copy(src_hbm.at[pl.ds(gb, GC)], sidx)
        pltpu.sync_copy(dst_hbm.at[pl.ds(gb, GC)], didx)
        pltpu.async_copy(table.at[sidx.at[0]], rows.at[0], gsem0)

        def body2(t, c2):
            j0 = 2 * t
            j1 = j0 + 1
            pltpu.async_copy(table.at[sidx.at[j1]], rows.at[1], gsem1)
            pltpu.make_async_copy(table.at[sidx.at[j0]], rows.at[0],
                                  gsem0).wait()
            pltpu.sync_copy(rows.at[0], acc.at[didx.at[j0]], add=True)

            @pl.when(t < GC // 2 - 1)
            def _():
                pltpu.async_copy(table.at[sidx.at[j0 + 2]], rows.at[0], gsem0)

            pltpu.make_async_copy(table.at[sidx.at[j1]], rows.at[1],
                                  gsem1).wait()
            pltpu.sync_copy(rows.at[1], acc.at[didx.at[j1]], add=True)
            return c2

        return lax.fori_loop(0, GC // 2, body2, c)

    lax.fori_loop(0, SP_CPT // GC, gbody, 0)
    plsc.subcore_barrier()

    @pl.when(sid < NS - 1)
    def _():
        pltpu.sync_copy(acc.at[pl.ds(rb, RPT)], s_out.at[cid, pl.ds(rb, RPT)])

    @pl.when(sid == NS - 1)
    def _():
        pltpu.sync_copy(acc.at[pl.ds(rb, RPT_LAST)],
                        s_out.at[cid, pl.ds(rb, RPT_LAST)])


_spmm_call = pl.kernel(
    _spmm_body,
    out_type=jax.ShapeDtypeStruct((NC, N, HF), jnp.float32),
    mesh=_MESH,
    compiler_params=_SC_PARAMS,
    scratch_types=[
        pltpu.VMEM((GC, CW), jnp.int32),
        pltpu.VMEM((GC, CW), jnp.int32),
        pltpu.VMEM((2, CW, HF), jnp.float32),
        pltpu.VMEM_SHARED((NR, HF), jnp.float32),
        pltpu.SemaphoreType.DMA,
        pltpu.SemaphoreType.DMA,
    ],
)


def _dinv_body(hist_ref, out_ref):
    s = jnp.sum(hist_ref[...], axis=0)
    out_ref[...] = lax.rsqrt(s[:N] + 1.0)[:, None]


_dinv_call = pl.pallas_call(
    _dinv_body,
    out_shape=jax.ShapeDtypeStruct((N, 1), jnp.float32),
)

R = 2000
NB = N // R


def _pre_body(x_ref, w_ref, dinv_ref, u_ref):
    z = jnp.dot(x_ref[...], w_ref[...], preferred_element_type=jnp.float32)
    u = dinv_ref[...] * z
    u_ref[0, :, :] = u[:, :HF]
    u_ref[1, :, :] = u[:, HF:]


_pre_call = pl.pallas_call(
    _pre_body,
    grid=(NB,),
    in_specs=[
        pl.BlockSpec((R, 10), lambda i: (i, 0)),
        pl.BlockSpec((10, H), lambda i: (0, 0)),
        pl.BlockSpec((R, 1), lambda i: (i, 0)),
    ],
    out_specs=pl.BlockSpec((NC, R, HF), lambda i: (0, i, 0)),
    out_shape=jax.ShapeDtypeStruct((NC, N, HF), jnp.float32),
)


def _mid_body(s_ref, dinv_ref, b_ref, w_ref, u_ref):
    dinv = dinv_ref[...]
    h0 = jnp.maximum(dinv * s_ref[0, :, :] + b_ref[0, :HF], 0.0)
    h1 = jnp.maximum(dinv * s_ref[1, :, :] + b_ref[0, HF:], 0.0)
    z = jnp.dot(h0, w_ref[:HF, :], preferred_element_type=jnp.float32)
    z = z + jnp.dot(h1, w_ref[HF:, :], preferred_element_type=jnp.float32)
    u = dinv * z
    u_ref[0, :, :] = u[:, :HF]
    u_ref[1, :, :] = u[:, HF:]


_mid_call = pl.pallas_call(
    _mid_body,
    grid=(NB,),
    in_specs=[
        pl.BlockSpec((NC, R, HF), lambda i: (0, i, 0)),
        pl.BlockSpec((R, 1), lambda i: (i, 0)),
        pl.BlockSpec((1, H), lambda i: (0, 0)),
        pl.BlockSpec((H, H), lambda i: (0, 0)),
    ],
    out_specs=pl.BlockSpec((NC, R, HF), lambda i: (0, i, 0)),
    out_shape=jax.ShapeDtypeStruct((NC, N, HF), jnp.float32),
)


def _final_body(s_ref, dinv_ref, b_ref, batch_ref, wl_ref, bl_ref, wl2_ref,
                bl2_ref, out_ref, pooled, counts):
    i = pl.program_id(0)

    @pl.when(i == 0)
    def _():
        pooled[...] = jnp.zeros_like(pooled)
        counts[...] = jnp.zeros_like(counts)

    dinv = dinv_ref[...]
    h = dinv * jnp.concatenate([s_ref[0, :, :], s_ref[1, :, :]], axis=1)
    h = h + b_ref[...]
    oh = (batch_ref[...] == lax.broadcasted_iota(jnp.int32, (R, G), 1))
    oh = oh.astype(jnp.float32)
    dn = (((0,), (0,)), ((), ()))
    hp = lax.Precision.HIGHEST
    pooled[...] += lax.dot_general(oh, h, dn, precision=hp,
                                   preferred_element_type=jnp.float32)
    counts[...] += lax.dot_general(oh, jnp.ones((R, 1), jnp.float32), dn,
                                   precision=hp,
                                   preferred_element_type=jnp.float32)

    @pl.when(i == NB - 1)
    def _():
        g = pooled[...] / jnp.maximum(counts[...], 1.0)
        y = jnp.dot(g, wl_ref[...], preferred_element_type=jnp.float32)
        y = y + bl_ref[...]
        mu = jnp.mean(y, axis=0, keepdims=True)
        var = jnp.mean((y - mu) ** 2, axis=0, keepdims=True)
        yr = jnp.maximum((y - mu) * lax.rsqrt(var + EPS), 0.0)
        out_ref[...] = (jnp.dot(yr, wl2_ref[...],
                                preferred_element_type=jnp.float32)
                        + bl2_ref[...])


_final_call = pl.pallas_call(
    _final_body,
    grid=(NB,),
    in_specs=[
        pl.BlockSpec((NC, R, HF), lambda i: (0, i, 0)),
        pl.BlockSpec((R, 1), lambda i: (i, 0)),
        pl.BlockSpec((1, H), lambda i: (0, 0)),
        pl.BlockSpec((R, 1), lambda i: (i, 0)),
        pl.BlockSpec((H, 256), lambda i: (0, 0)),
        pl.BlockSpec((1, 256), lambda i: (0, 0)),
        pl.BlockSpec((256, 2), lambda i: (0, 0)),
        pl.BlockSpec((1, 2), lambda i: (0, 0)),
    ],
    out_specs=pl.BlockSpec((G, 2), lambda i: (0, 0)),
    out_shape=jax.ShapeDtypeStruct((G, 2), jnp.float32),
    scratch_shapes=[
        pltpu.VMEM((G, H), jnp.float32),
        pltpu.VMEM((G, 1), jnp.float32),
    ],
)


def kernel(x, edge_index, edge_weight, batch, W1, b1, W2, b2, W3, b3, W4, b4,
           Wl, bl, Wl2, bl2):
    ei = edge_index.astype(jnp.int32)
    src = ei[0]
    dst = ei[1]
    pad = EP - E
    src_p = jnp.concatenate([src, jnp.zeros((pad,), jnp.int32)])
    src_p = src_p.reshape(NCHUNK_PAD, CW)
    dst_p = jnp.concatenate([dst, jnp.full((pad,), N, jnp.int32)])
    dst_p = dst_p.reshape(NCHUNK_PAD, CW)
    batch2 = batch.astype(jnp.int32).reshape(N, 1)

    hist = _deg_call(dst_p)
    dinv = _dinv_call(hist)
    u = _pre_call(x, W1, dinv)
    for b_l, W_next in ((b1, W2), (b2, W3), (b3, W4)):
        s = _spmm_call(u, src_p, dst_p)
        u = _mid_call(s, dinv, b_l.reshape(1, H), W_next)
    s = _spmm_call(u, src_p, dst_p)
    return _final_call(s, dinv, b4.reshape(1, H), batch2, Wl,
                       bl.reshape(1, 256), Wl2, bl2.reshape(1, 2))
